# Initial kernel scaffold; baseline (speedup 1.0000x reference)
import jax
import jax.numpy as jnp
from jax.experimental import pallas as pl


def kernel(x):
    raise NotImplementedError("write your pallas kernel here")


# speedup vs baseline: 4.2873x; 4.2873x over previous
import jax
import jax.numpy as jnp
from jax import lax
from jax.experimental import pallas as pl
from jax.experimental.pallas import tpu as pltpu

_TOPK = 100


def _topk_body(x_ref, o_ref, s_ref):
    _NEG = float("-inf")
    _BIG = 2**30
    hm = x_ref[0, :80, :, :]
    s_ref[...] = hm
    m = jnp.max(hm, axis=2)
    a_io = lax.broadcasted_iota(jnp.int32, (80, 128), 0)
    b_io = lax.broadcasted_iota(jnp.int32, (80, 128), 1)
    ridx = a_io * 128 + b_io
    lane1 = lax.broadcasted_iota(jnp.int32, (1, 128), 1)
    row2d = lax.broadcasted_iota(jnp.int32, (128, 128), 0)
    lane2d = lax.broadcasted_iota(jnp.int32, (128, 128), 1)
    wh0 = x_ref[0, 80, :, :]
    wh1 = x_ref[0, 81, :, :]
    reg0 = x_ref[0, 82, :, :]
    reg1 = x_ref[0, 83, :, :]

    def step(k, carry):
        m, o0, o1, o2, o3, o4, o5 = carry
        gmax = jnp.max(m)
        rstar = jnp.min(jnp.where(m == gmax, ridx, _BIG))
        c = rstar >> 7
        y = rstar & 127
        plane = s_ref[c]
        rowsel = row2d == y
        rowvals = jnp.max(jnp.where(rowsel, plane, _NEG), axis=0, keepdims=True)
        lstar = jnp.min(jnp.where(rowvals == gmax, lane1, _BIG))
        elsel = rowsel & (lane2d == lstar)
        s_ref[c] = jnp.where(elsel, _NEG, plane)
        newmax = jnp.max(jnp.where(lane1 == lstar, _NEG, rowvals))
        m = jnp.where(ridx == rstar, newmax, m)
        spat = y * 128 + lstar
        gidx = rstar * 128 + lstar
        xo = jnp.sum(jnp.where(elsel, reg0, 0.0))
        yo = jnp.sum(jnp.where(elsel, reg1, 0.0))
        bw = jnp.sum(jnp.where(elsel, wh0, 0.0))
        bh = jnp.sum(jnp.where(elsel, wh1, 0.0))
        cls = gidx.astype(jnp.float32) / jnp.float32(16384.0)
        cy = spat.astype(jnp.float32) / jnp.float32(128.0) + yo
        cx = lstar.astype(jnp.float32) + xo
        hw = 0.5 * bw
        hh = 0.5 * bh
        s4 = jnp.float32(4.0)
        km = lane1 == k
        o0 = jnp.where(km, (cx - hw) * s4, o0)
        o1 = jnp.where(km, (cy - hh) * s4, o1)
        o2 = jnp.where(km, (cx + hw) * s4, o2)
        o3 = jnp.where(km, (cy + hh) * s4, o3)
        o4 = jnp.where(km, cls, o4)
        o5 = jnp.where(km, gmax, o5)
        return (m, o0, o1, o2, o3, o4, o5)

    z = jnp.zeros((1, 128), jnp.float32)
    carry = lax.fori_loop(0, _TOPK, step, (m, z, z, z, z, z, z))
    o_ref[0] = jnp.concatenate(carry[1:], axis=0)


def _build(interpret=False):
    return pl.pallas_call(
        _topk_body,
        grid=(16,),
        in_specs=[pl.BlockSpec((1, 84, 128, 128), lambda b: (b, 0, 0, 0))],
        out_specs=pl.BlockSpec((1, 6, 128), lambda b: (b, 0, 0)),
        out_shape=jax.ShapeDtypeStruct((16, 6, 128), jnp.float32),
        scratch_shapes=[pltpu.VMEM((80, 128, 128), jnp.float32)],
        interpret=interpret,
    )


@jax.jit
def kernel(x):
    rows = _build()(x)
    return jnp.transpose(rows, (0, 2, 1))[:, :_TOPK, :]


# speedup vs baseline: 4.5521x; 1.0618x over previous
import jax
import jax.numpy as jnp
from jax import lax
from jax.experimental import pallas as pl
from jax.experimental.pallas import tpu as pltpu

_TOPK = 100


def _topk_body(x_ref, o_ref, s_ref):
    _NEG = float("-inf")
    _BIG = 2**30
    hm = x_ref[0, :80, :, :]
    s_ref[...] = hm.reshape(10240, 128)
    m = jnp.max(hm, axis=2)
    a_io = lax.broadcasted_iota(jnp.int32, (80, 128), 0)
    b_io = lax.broadcasted_iota(jnp.int32, (80, 128), 1)
    ridx = a_io * 128 + b_io
    lane1 = lax.broadcasted_iota(jnp.int32, (1, 128), 1)

    def step(k, carry):
        m, oi, os = carry
        gmax = jnp.max(m)
        rstar = jnp.min(jnp.where(m == gmax, ridx, _BIG))
        row = s_ref[pl.ds(rstar, 1), :]
        lstar = jnp.min(jnp.where(row == gmax, lane1, _BIG))
        newrow = jnp.where(lane1 == lstar, _NEG, row)
        s_ref[pl.ds(rstar, 1), :] = newrow
        m = jnp.where(ridx == rstar, jnp.max(newrow), m)
        km = lane1 == k
        oi = jnp.where(km, rstar * 128 + lstar, oi)
        os = jnp.where(km, gmax, os)
        return (m, oi, os)

    zi = jnp.zeros((1, 128), jnp.int32)
    zs = jnp.zeros((1, 128), jnp.float32)
    _, idx, score = lax.fori_loop(0, _TOPK, step, (m, zi, zs))

    f32 = jnp.float32
    y = (idx >> 7) & 127
    xl = idx & 127
    spat = idx & 16383
    sub2d = lax.broadcasted_iota(jnp.int32, (128, 128), 0)
    by = (sub2d == y).astype(f32)
    bx = (sub2d == xl).astype(f32)
    dn = (((0,), (0,)), ((), ()))

    def gather_ch(ch):
        p = lax.dot_general(ch, by, dn, preferred_element_type=f32)
        return jnp.sum(p * bx, axis=0, keepdims=True)

    bw = gather_ch(x_ref[0, 80, :, :])
    bh = gather_ch(x_ref[0, 81, :, :])
    xo = gather_ch(x_ref[0, 82, :, :])
    yo = gather_ch(x_ref[0, 83, :, :])
    cls = idx.astype(f32) / f32(16384.0)
    cy = spat.astype(f32) / f32(128.0) + yo
    cx = xl.astype(f32) + xo
    hw = 0.5 * bw
    hh = 0.5 * bh
    s4 = f32(4.0)
    o_ref[0] = jnp.concatenate(
        [(cx - hw) * s4, (cy - hh) * s4, (cx + hw) * s4, (cy + hh) * s4, cls, score],
        axis=0,
    )


def _build(interpret=False):
    return pl.pallas_call(
        _topk_body,
        grid=(16,),
        in_specs=[pl.BlockSpec((1, 84, 128, 128), lambda b: (b, 0, 0, 0))],
        out_specs=pl.BlockSpec((1, 6, 128), lambda b: (b, 0, 0)),
        out_shape=jax.ShapeDtypeStruct((16, 6, 128), jnp.float32),
        scratch_shapes=[pltpu.VMEM((10240, 128), jnp.float32)],
        interpret=interpret,
    )


@jax.jit
def kernel(x):
    rows = _build()(x)
    return jnp.transpose(rows, (0, 2, 1))[:, :_TOPK, :]
